# Initial kernel scaffold; baseline (speedup 1.0000x reference)
import jax
import jax.numpy as jnp
from jax.experimental import pallas as pl


def kernel(x, edge_index, edge_attr, W, b):
    raise NotImplementedError("write your pallas kernel here")


# speedup vs baseline: 6.8764x; 6.8764x over previous
import functools

import jax
import jax.numpy as jnp
from jax import lax
from jax.experimental import pallas as pl
from jax.experimental.pallas import tpu as pltpu
from jax.experimental.pallas import tpu_sc as plsc

N_NODES = 100000
N_EDGES = 3200000
D_EDGE = 16
D_FEAT = 128
SLAB = 128
GS = 8
GEDGES = SLAB * GS
NGROUP = N_EDGES // GEDGES
NW = 32
BASE_G = NGROUP // NW
EXTRA_G = NGROUP - BASE_G * NW
NRB = N_NODES // 8
BASE_R = NRB // 16
EXTRA_R = NRB - BASE_R * 16


def _sc_aggregate(col2d, edge_attr, zsum, zcnt, ones):
    mesh = plsc.VectorSubcoreMesh(core_axis_name="c", subcore_axis_name="s")

    @functools.partial(
        pl.kernel,
        mesh=mesh,
        out_type=[
            jax.ShapeDtypeStruct((2, N_NODES, D_EDGE), jnp.float32),
            jax.ShapeDtypeStruct((2, N_NODES), jnp.float32),
        ],
        scratch_types=[
            pltpu.VMEM_SHARED((N_NODES, D_EDGE), jnp.float32),
            pltpu.VMEM_SHARED((N_NODES,), jnp.float32),
            pltpu.VMEM((GS, SLAB), jnp.int32),
            pltpu.VMEM((GEDGES, D_EDGE), jnp.float32),
            pltpu.VMEM((SLAB,), jnp.float32),
        ],
        compiler_params=pltpu.CompilerParams(use_tc_tiling_on_sc=False),
    )
    def k(col_h, ea_h, zs_h, zc_h, ones_h, sums_o, cnts_o,
          sums_s, cnts_s, idx_v, ea_v, ones_v):
        cid = lax.axis_index("c")
        sid = lax.axis_index("s")
        wid = sid * 2 + cid
        r0 = (sid * BASE_R + jnp.minimum(sid, EXTRA_R)) * 8

        def zero_rows(nrows):
            pltpu.sync_copy(zs_h.at[pl.ds(r0, nrows)],
                            sums_s.at[pl.ds(r0, nrows)])
            pltpu.sync_copy(zc_h.at[pl.ds(r0, nrows)],
                            cnts_s.at[pl.ds(r0, nrows)])

        def drain_rows(nrows):
            pltpu.sync_copy(sums_s.at[pl.ds(r0, nrows)],
                            sums_o.at[cid, pl.ds(r0, nrows)])
            pltpu.sync_copy(cnts_s.at[pl.ds(r0, nrows)],
                            cnts_o.at[cid, pl.ds(r0, nrows)])

        @pl.when(sid < EXTRA_R)
        def _():
            zero_rows((BASE_R + 1) * 8)

        @pl.when(sid >= EXTRA_R)
        def _():
            zero_rows(BASE_R * 8)

        pltpu.sync_copy(ones_h, ones_v)
        plsc.subcore_barrier()

        start = wid * BASE_G + jnp.minimum(wid, EXTRA_G)
        ngroups = BASE_G + (wid < EXTRA_G).astype(jnp.int32)

        def step(i, carry):
            g = start + i
            pltpu.sync_copy(col_h.at[pl.ds(g * GS, GS)], idx_v)
            pltpu.sync_copy(ea_h.at[pl.ds(g * GEDGES, GEDGES)], ea_v)
            for j in range(GS):
                pltpu.sync_copy(ea_v.at[pl.ds(j * SLAB, SLAB)],
                                sums_s.at[idx_v.at[j]], add=True)
                pltpu.sync_copy(ones_v, cnts_s.at[idx_v.at[j]], add=True)
            return carry

        lax.fori_loop(0, ngroups, step, 0)
        plsc.subcore_barrier()

        @pl.when(sid < EXTRA_R)
        def _():
            drain_rows((BASE_R + 1) * 8)

        @pl.when(sid >= EXTRA_R)
        def _():
            drain_rows(BASE_R * 8)

    return k(col2d, edge_attr, zsum, zcnt, ones)


ROWS_TC = 2000


def _tc_update(x, sums, cnts, wxt, wat, b2):
    def body(x_r, s_r, c_r, wx_r, wa_r, b_r, o_r):
        s = s_r[0] + s_r[1]
        c = c_r[0].reshape(ROWS_TC, 1) + c_r[1].reshape(ROWS_TC, 1)
        agg = s / jnp.maximum(c, 1.0)
        o_r[...] = (
            jnp.dot(x_r[...], wx_r[...], preferred_element_type=jnp.float32)
            + jnp.dot(agg, wa_r[...], preferred_element_type=jnp.float32)
            + b_r[...]
        )

    return pl.pallas_call(
        body,
        grid=(N_NODES // ROWS_TC,),
        in_specs=[
            pl.BlockSpec((ROWS_TC, D_FEAT), lambda i: (i, 0)),
            pl.BlockSpec((2, ROWS_TC, D_EDGE), lambda i: (0, i, 0)),
            pl.BlockSpec((2, ROWS_TC, 1), lambda i: (0, i, 0)),
            pl.BlockSpec((D_FEAT, D_FEAT), lambda i: (0, 0)),
            pl.BlockSpec((D_EDGE, D_FEAT), lambda i: (0, 0)),
            pl.BlockSpec((1, D_FEAT), lambda i: (0, 0)),
        ],
        out_specs=pl.BlockSpec((ROWS_TC, D_FEAT), lambda i: (i, 0)),
        out_shape=jax.ShapeDtypeStruct((N_NODES, D_FEAT), jnp.float32),
    )(x, sums, cnts, wxt, wat, b2)


def kernel(x, edge_index, edge_attr, W, b):
    col2d = edge_index[1].reshape(NGROUP * GS, SLAB)
    zsum = jnp.zeros((N_NODES, D_EDGE), jnp.float32)
    zcnt = jnp.zeros((N_NODES,), jnp.float32)
    ones = jnp.ones((SLAB,), jnp.float32)
    sums, cnts = _sc_aggregate(col2d, edge_attr, zsum, zcnt, ones)
    wxt = W[:, :D_FEAT].T
    wat = W[:, D_FEAT:].T
    b2 = b.reshape(1, D_FEAT)
    return _tc_update(x, sums, cnts.reshape(2, N_NODES, 1), wxt, wat, b2)


# speedup vs baseline: 6.8796x; 1.0005x over previous
import functools

import jax
import jax.numpy as jnp
from jax import lax
from jax.experimental import pallas as pl
from jax.experimental.pallas import tpu as pltpu
from jax.experimental.pallas import tpu_sc as plsc

N_NODES = 100000
N_EDGES = 3200000
D_EDGE = 16
D_FEAT = 128
SLAB = 128
GS = 8
GEDGES = SLAB * GS
NGROUP = N_EDGES // GEDGES
NW = 32
BASE_G = NGROUP // NW
EXTRA_G = NGROUP - BASE_G * NW
NRB = N_NODES // 8
BASE_R = NRB // 16
EXTRA_R = NRB - BASE_R * 16


def _sc_aggregate(col2d, edge_attr, zsum, zcnt, ones):
    mesh = plsc.VectorSubcoreMesh(core_axis_name="c", subcore_axis_name="s")

    @functools.partial(
        pl.kernel,
        mesh=mesh,
        out_type=[
            jax.ShapeDtypeStruct((2, N_NODES, D_EDGE), jnp.float32),
            jax.ShapeDtypeStruct((2, N_NODES), jnp.float32),
        ],
        scratch_types=[
            pltpu.VMEM_SHARED((N_NODES, D_EDGE), jnp.float32),
            pltpu.VMEM_SHARED((N_NODES,), jnp.float32),
            pltpu.VMEM((GEDGES,), jnp.int32),
            pltpu.VMEM((GEDGES, D_EDGE), jnp.float32),
            pltpu.VMEM((SLAB,), jnp.float32),
        ],
        compiler_params=pltpu.CompilerParams(use_tc_tiling_on_sc=False),
    )
    def k(col_h, ea_h, zs_h, zc_h, ones_h, sums_o, cnts_o,
          sums_s, cnts_s, idx_v, ea_v, ones_v):
        cid = lax.axis_index("c")
        sid = lax.axis_index("s")
        wid = sid * 2 + cid
        r0 = (sid * BASE_R + jnp.minimum(sid, EXTRA_R)) * 8

        def zero_rows(nrows):
            pltpu.sync_copy(zs_h.at[pl.ds(r0, nrows)],
                            sums_s.at[pl.ds(r0, nrows)])
            pltpu.sync_copy(zc_h.at[pl.ds(r0, nrows)],
                            cnts_s.at[pl.ds(r0, nrows)])

        def drain_rows(nrows):
            pltpu.sync_copy(sums_s.at[pl.ds(r0, nrows)],
                            sums_o.at[cid, pl.ds(r0, nrows)])
            pltpu.sync_copy(cnts_s.at[pl.ds(r0, nrows)],
                            cnts_o.at[cid, pl.ds(r0, nrows)])

        @pl.when(sid < EXTRA_R)
        def _():
            zero_rows((BASE_R + 1) * 8)

        @pl.when(sid >= EXTRA_R)
        def _():
            zero_rows(BASE_R * 8)

        pltpu.sync_copy(ones_h, ones_v)
        plsc.subcore_barrier()

        start = wid * BASE_G + jnp.minimum(wid, EXTRA_G)
        ngroups = BASE_G + (wid < EXTRA_G).astype(jnp.int32)

        def step(i, carry):
            g = start + i
            pltpu.sync_copy(col_h.at[1, pl.ds(g * GEDGES, GEDGES)], idx_v)
            pltpu.sync_copy(ea_h.at[pl.ds(g * GEDGES, GEDGES)], ea_v)
            for j in range(GS):
                ix = idx_v.at[pl.ds(j * SLAB, SLAB)]
                pltpu.sync_copy(ea_v.at[pl.ds(j * SLAB, SLAB)],
                                sums_s.at[ix], add=True)
                pltpu.sync_copy(ones_v, cnts_s.at[ix], add=True)
            return carry

        lax.fori_loop(0, ngroups, step, 0)
        plsc.subcore_barrier()

        @pl.when(sid < EXTRA_R)
        def _():
            drain_rows((BASE_R + 1) * 8)

        @pl.when(sid >= EXTRA_R)
        def _():
            drain_rows(BASE_R * 8)

    return k(col2d, edge_attr, zsum, zcnt, ones)


ROWS_TC = 2000


def _tc_update(x, sums, cnts, wxt, wat, b2):
    def body(x_r, s_r, c_r, wx_r, wa_r, b_r, o_r):
        s = s_r[0] + s_r[1]
        c = c_r[0].reshape(ROWS_TC, 1) + c_r[1].reshape(ROWS_TC, 1)
        agg = s / jnp.maximum(c, 1.0)
        o_r[...] = (
            jnp.dot(x_r[...], wx_r[...], preferred_element_type=jnp.float32)
            + jnp.dot(agg, wa_r[...], preferred_element_type=jnp.float32)
            + b_r[...]
        )

    return pl.pallas_call(
        body,
        grid=(N_NODES // ROWS_TC,),
        in_specs=[
            pl.BlockSpec((ROWS_TC, D_FEAT), lambda i: (i, 0)),
            pl.BlockSpec((2, ROWS_TC, D_EDGE), lambda i: (0, i, 0)),
            pl.BlockSpec((2, ROWS_TC, 1), lambda i: (0, i, 0)),
            pl.BlockSpec((D_FEAT, D_FEAT), lambda i: (0, 0)),
            pl.BlockSpec((D_EDGE, D_FEAT), lambda i: (0, 0)),
            pl.BlockSpec((1, D_FEAT), lambda i: (0, 0)),
        ],
        out_specs=pl.BlockSpec((ROWS_TC, D_FEAT), lambda i: (i, 0)),
        out_shape=jax.ShapeDtypeStruct((N_NODES, D_FEAT), jnp.float32),
    )(x, sums, cnts, wxt, wat, b2)


def kernel(x, edge_index, edge_attr, W, b):
    zsum = jnp.zeros((N_NODES, D_EDGE), jnp.float32)
    zcnt = jnp.zeros((N_NODES,), jnp.float32)
    ones = jnp.ones((SLAB,), jnp.float32)
    sums, cnts = _sc_aggregate(edge_index, edge_attr, zsum, zcnt, ones)
    wxt = W[:, :D_FEAT].T
    wat = W[:, D_FEAT:].T
    b2 = b.reshape(1, D_FEAT)
    return _tc_update(x, sums, cnts.reshape(2, N_NODES, 1), wxt, wat, b2)


# speedup vs baseline: 7.2789x; 1.0580x over previous
import functools

import jax
import jax.numpy as jnp
from jax import lax
from jax.experimental import pallas as pl
from jax.experimental.pallas import tpu as pltpu
from jax.experimental.pallas import tpu_sc as plsc

N_NODES = 100000
N_EDGES = 3200000
D_EDGE = 16
D_FEAT = 128
SLAB = 128
GS = 8
GEDGES = SLAB * GS
NGROUP = N_EDGES // GEDGES
NW = 32
BASE_G = NGROUP // NW
EXTRA_G = NGROUP - BASE_G * NW
NRB = N_NODES // 8
BASE_R = NRB // 16
EXTRA_R = NRB - BASE_R * 16


def _sc_aggregate(col2d, edge_attr, zsum, zcnt, ones):
    mesh = plsc.VectorSubcoreMesh(core_axis_name="c", subcore_axis_name="s")

    @functools.partial(
        pl.kernel,
        mesh=mesh,
        out_type=[
            jax.ShapeDtypeStruct((2, N_NODES, D_EDGE), jnp.float32),
            jax.ShapeDtypeStruct((2, N_NODES), jnp.float32),
        ],
        scratch_types=[
            pltpu.VMEM_SHARED((N_NODES, D_EDGE), jnp.float32),
            pltpu.VMEM_SHARED((N_NODES,), jnp.float32),
            pltpu.VMEM((GEDGES,), jnp.int32),
            pltpu.VMEM((GEDGES, D_EDGE), jnp.float32),
            pltpu.VMEM((SLAB,), jnp.float32),
        ],
        compiler_params=pltpu.CompilerParams(use_tc_tiling_on_sc=False),
    )
    def k(col_h, ea_h, zs_h, zc_h, ones_h, sums_o, cnts_o,
          sums_s, cnts_s, idx_v, ea_v, ones_v):
        cid = lax.axis_index("c")
        sid = lax.axis_index("s")
        wid = sid * 2 + cid
        r0 = (sid * BASE_R + jnp.minimum(sid, EXTRA_R)) * 8

        def zero_rows(nrows):
            pltpu.sync_copy(zs_h.at[pl.ds(r0, nrows)],
                            sums_s.at[pl.ds(r0, nrows)])
            pltpu.sync_copy(zc_h.at[pl.ds(r0, nrows)],
                            cnts_s.at[pl.ds(r0, nrows)])

        def drain_rows(nrows):
            pltpu.sync_copy(sums_s.at[pl.ds(r0, nrows)],
                            sums_o.at[cid, pl.ds(r0, nrows)])
            pltpu.sync_copy(cnts_s.at[pl.ds(r0, nrows)],
                            cnts_o.at[cid, pl.ds(r0, nrows)])

        @pl.when(sid < EXTRA_R)
        def _():
            zero_rows((BASE_R + 1) * 8)

        @pl.when(sid >= EXTRA_R)
        def _():
            zero_rows(BASE_R * 8)

        pltpu.sync_copy(ones_h, ones_v)
        plsc.subcore_barrier()

        start = wid * BASE_G + jnp.minimum(wid, EXTRA_G)
        ngroups = BASE_G + (wid < EXTRA_G).astype(jnp.int32)

        def step(i, carry):
            g = start + i
            pltpu.sync_copy(col_h.at[1, pl.ds(g * GEDGES, GEDGES)], idx_v)
            pltpu.sync_copy(ea_h.at[pl.ds(g * GEDGES, GEDGES)], ea_v)
            for j in range(GS):
                ix = idx_v.at[pl.ds(j * SLAB, SLAB)]
                pltpu.sync_copy(ea_v.at[pl.ds(j * SLAB, SLAB)],
                                sums_s.at[ix], add=True)
                pltpu.sync_copy(ones_v, cnts_s.at[ix], add=True)
            return carry

        lax.fori_loop(0, ngroups, step, 0)
        plsc.subcore_barrier()

        @pl.when(sid < EXTRA_R)
        def _():
            drain_rows((BASE_R + 1) * 8)

        @pl.when(sid >= EXTRA_R)
        def _():
            drain_rows(BASE_R * 8)

    return k(col2d, edge_attr, zsum, zcnt, ones)


ROWS_TC = 2000
PR = ROWS_TC // 8
NPR = N_NODES // 8


def _tc_update(x, sums_p, cnts_p, wxt, wab, b2):
    def body(x_r, s_r, c_r, wx_r, wab_r, b_r, o_r):
        s = s_r[0, 0] + s_r[1, 0]
        y = jnp.dot(s, wab_r[...],
                    preferred_element_type=jnp.float32)
        y = y.reshape(ROWS_TC, D_FEAT)
        c = c_r[0] + c_r[1]
        o_r[...] = (
            jnp.dot(x_r[...], wx_r[...], preferred_element_type=jnp.float32)
            + y / jnp.maximum(c, 1.0)
            + b_r[...]
        )

    return pl.pallas_call(
        body,
        grid=(N_NODES // ROWS_TC,),
        in_specs=[
            pl.BlockSpec((ROWS_TC, D_FEAT), lambda i: (i, 0)),
            pl.BlockSpec((2, 1, PR, 128), lambda i: (0, i, 0, 0)),
            pl.BlockSpec((2, ROWS_TC, 1), lambda i: (0, i, 0)),
            pl.BlockSpec((D_FEAT, D_FEAT), lambda i: (0, 0)),
            pl.BlockSpec((D_FEAT, 8 * D_FEAT), lambda i: (0, 0)),
            pl.BlockSpec((1, D_FEAT), lambda i: (0, 0)),
        ],
        out_specs=pl.BlockSpec((ROWS_TC, D_FEAT), lambda i: (i, 0)),
        out_shape=jax.ShapeDtypeStruct((N_NODES, D_FEAT), jnp.float32),
    )(x, sums_p, cnts_p, wxt, wab, b2)


def kernel(x, edge_index, edge_attr, W, b):
    zsum = jnp.zeros((N_NODES, D_EDGE), jnp.float32)
    zcnt = jnp.zeros((N_NODES,), jnp.float32)
    ones = jnp.ones((SLAB,), jnp.float32)
    sums, cnts = _sc_aggregate(edge_index, edge_attr, zsum, zcnt, ones)
    nblk = N_NODES // ROWS_TC
    sums_p = sums.reshape(2, nblk, PR, 128)
    cnts_p = cnts.reshape(2, N_NODES, 1)
    wxt = W[:, :D_FEAT].T
    wab = jnp.kron(jnp.eye(8, dtype=jnp.float32), W[:, D_FEAT:].T)
    b2 = b.reshape(1, D_FEAT)
    return _tc_update(x, sums_p, cnts_p, wxt, wab, b2)
